# baseline (speedup 1.0000x reference)
import jax
import jax.numpy as jnp
from jax import lax
from jax.experimental import pallas as pl
from jax.experimental.pallas import tpu as pltpu
from jax.experimental.pallas import tpu_sc as plsc

_D = 64
_B = 4096 * 200
_NC, _NS = 2, 16
_NW = _NC * _NS
_BPW = _B // _NW
_CHUNK = 128
_NCHUNK = _BPW // _CHUNK


def _gather_body(table_hbm, idx_hbm, out_hbm, idx_v, rows_v, sem):
    wid = lax.axis_index("s") * _NC + lax.axis_index("c")
    pltpu.sync_copy(idx_hbm.at[pl.ds(wid * _NCHUNK, _NCHUNK)], idx_v)
    base = wid * _BPW

    def step(c, carry):
        pltpu.async_copy(table_hbm.at[idx_v.at[c]], rows_v, sem).wait()
        pltpu.sync_copy(rows_v, out_hbm.at[pl.ds(base + c * _CHUNK, _CHUNK)])
        return carry

    lax.fori_loop(0, _NCHUNK, step, 0)


@jax.jit
def kernel(x, weight):
    batch, hist = x.shape
    idx = x.reshape(_NW * _NCHUNK, _CHUNK).astype(jnp.int32)
    mesh = plsc.VectorSubcoreMesh(core_axis_name="c", subcore_axis_name="s")
    out = pl.kernel(
        _gather_body,
        out_type=jax.ShapeDtypeStruct((_B, _D), jnp.float32),
        mesh=mesh,
        scratch_types=[
            pltpu.VMEM((_NCHUNK, _CHUNK), jnp.int32),
            pltpu.VMEM((_CHUNK, _D), jnp.float32),
            pltpu.SemaphoreType.DMA,
        ],
        compiler_params=pltpu.CompilerParams(use_tc_tiling_on_sc=False),
    )(weight, idx)
    return out.reshape(batch, hist, _D)


# speedup vs baseline: 1.1146x; 1.1146x over previous
import jax
import jax.numpy as jnp
from jax import lax
from jax.experimental import pallas as pl
from jax.experimental.pallas import tpu as pltpu
from jax.experimental.pallas import tpu_sc as plsc

_D = 64
_B = 4096 * 200
_NC, _NS = 2, 16
_NW = _NC * _NS
_BPW = _B // _NW
_CHUNK = 128
_NCHUNK = _BPW // _CHUNK


_NBUF = 4
_NSTEPS = _NCHUNK // _NBUF


def _gather_body(table_hbm, idx_hbm, out_hbm, idx_v, rows_v, *sems):
    g_sems, o_sems = sems[:_NBUF], sems[_NBUF:]
    wid = lax.axis_index("s") * _NC + lax.axis_index("c")
    pltpu.sync_copy(idx_hbm.at[pl.ds(wid * _NCHUNK, _NCHUNK)], idx_v)
    base = wid * _BPW

    def fire_gather(c, b):
        pltpu.async_copy(table_hbm.at[idx_v.at[c]], rows_v.at[b], g_sems[b])

    def fire_out(c, b):
        pltpu.async_copy(
            rows_v.at[b], out_hbm.at[pl.ds(base + c * _CHUNK, _CHUNK)], o_sems[b]
        )

    def drain_gather(c, b):
        pltpu.make_async_copy(
            table_hbm.at[idx_v.at[c]], rows_v.at[b], g_sems[b]
        ).wait()

    def drain_out(c, b):
        pltpu.make_async_copy(
            rows_v.at[b], out_hbm.at[pl.ds(base + c * _CHUNK, _CHUNK)], o_sems[b]
        ).wait()

    for b in range(_NBUF):
        fire_gather(b, b)

    def step(s, carry):
        c0 = s * _NBUF
        for b in range(_NBUF):
            drain_gather(c0 + b, b)
            fire_out(c0 + b, b)
        for b in range(_NBUF):
            drain_out(c0 + b, b)
            fire_gather(c0 + _NBUF + b, b)
        return carry

    lax.fori_loop(0, _NSTEPS - 1, step, 0)

    c0 = (_NSTEPS - 1) * _NBUF
    for b in range(_NBUF):
        drain_gather(c0 + b, b)
        fire_out(c0 + b, b)
    for b in range(_NBUF):
        drain_out(c0 + b, b)


@jax.jit
def kernel(x, weight):
    batch, hist = x.shape
    idx = x.reshape(_NW * _NCHUNK, _CHUNK).astype(jnp.int32)
    mesh = plsc.VectorSubcoreMesh(core_axis_name="c", subcore_axis_name="s")
    out = pl.kernel(
        _gather_body,
        out_type=jax.ShapeDtypeStruct((_B, _D), jnp.float32),
        mesh=mesh,
        scratch_types=[
            pltpu.VMEM((_NCHUNK, _CHUNK), jnp.int32),
            pltpu.VMEM((_NBUF, _CHUNK, _D), jnp.float32),
        ] + [pltpu.SemaphoreType.DMA] * (2 * _NBUF),
        compiler_params=pltpu.CompilerParams(use_tc_tiling_on_sc=False),
    )(weight, idx)
    return out.reshape(batch, hist, _D)
